# baseline (speedup 1.0000x reference)
import functools

import jax
import jax.numpy as jnp
from jax import lax
from jax.experimental import pallas as pl
from jax.experimental.pallas import tpu as pltpu
from jax.experimental.pallas import tpu_sc as plsc

NC = 2
NS = 16
L = 16
NW = NC * NS

N_POINTS = 100000
N_EDGES = 3200000
EPT = N_EDGES // NW
CH = 400
NCHUNK = EPT // CH
RPS = N_POINTS // NS
PADW = 8


@functools.partial(
    pl.kernel,
    out_type=(
        jax.ShapeDtypeStruct((N_EDGES, 3), jnp.float32),
        jax.ShapeDtypeStruct((N_EDGES, 3), jnp.float32),
    ),
    mesh=plsc.VectorSubcoreMesh(core_axis_name="c", subcore_axis_name="s"),
    scratch_types=[
        pltpu.VMEM((CH,), jnp.int32),
        pltpu.VMEM((CH,), jnp.int32),
        pltpu.VMEM((CH,), jnp.int32),
        pltpu.VMEM((CH,), jnp.int32),
        pltpu.VMEM((CH, PADW), jnp.float32),
        pltpu.VMEM((CH, PADW), jnp.float32),
        pltpu.VMEM((CH, PADW), jnp.float32),
        pltpu.VMEM((CH, PADW), jnp.float32),
        pltpu.VMEM((CH, 3), jnp.float32),
        pltpu.VMEM((L,), jnp.float32),
        pltpu.VMEM_SHARED((N_POINTS, PADW), jnp.float32),
        pltpu.SemaphoreType.DMA,
        pltpu.SemaphoreType.DMA,
        pltpu.SemaphoreType.DMA,
        pltpu.SemaphoreType.DMA,
    ],
    compiler_params=pltpu.CompilerParams(
        use_tc_tiling_on_sc=False, needs_layout_passes=False
    ),
)
def _graph_gather(tab_hbm, eidx_hbm, mul_hbm, p1_hbm, p2_hbm,
                  idx_a, idx_b, idx_c, idx_d,
                  rows_a, rows_b, rows_c, rows_d, rows3, mulv, tab_sh,
                  sem_a, sem_b, sem_c, sem_d):
    cid = lax.axis_index("c")
    sid = lax.axis_index("s")
    wid = sid * NC + cid

    pltpu.sync_copy(tab_hbm.at[pl.ds(sid * RPS, RPS)],
                    tab_sh.at[pl.ds(sid * RPS, RPS)])
    pltpu.sync_copy(mul_hbm, mulv)
    plsc.subcore_barrier()
    m = mulv[...]

    lanes = lax.iota(jnp.int32, 16)
    pats = []
    for j in range(3):
        p = j * 16 + lanes
        pats.append((p // 3, p - (p // 3) * 3))

    def compact_to(rows8):
        def compact(t, carry2):
            for j in range(3):
                r0, c0 = pats[j]
                r = t * 16 + r0
                v = plsc.load_gather(rows8, [r, c0])
                plsc.store_scatter(rows3, [r, c0], v * m)
            return carry2

        lax.fori_loop(0, CH // 16, compact, 0)

    def start_gather(e, g, idx_v, rows8, sem):
        base = wid * EPT + g * CH
        pltpu.sync_copy(eidx_hbm.at[pl.ds(e * N_EDGES + base, CH)], idx_v)
        pltpu.async_copy(tab_sh.at[idx_v], rows8, sem)

    def finish(idx_v, rows8, sem, g, out_hbm):
        pltpu.make_async_copy(tab_sh.at[idx_v], rows8, sem).wait()
        compact_to(rows8)
        pltpu.sync_copy(rows3, out_hbm.at[pl.ds(wid * EPT + g * CH, CH)])

    def body(g, carry):
        start_gather(0, g, idx_a, rows_a, sem_a)
        start_gather(1, g, idx_b, rows_b, sem_b)
        finish(idx_a, rows_a, sem_a, g, p1_hbm)
        finish(idx_b, rows_b, sem_b, g, p2_hbm)
        return carry

    lax.fori_loop(0, NCHUNK, body, 0)


def kernel(coords, edge_index, mul):
    tab8 = jnp.pad(coords, ((0, 0), (0, PADW - 3)))
    eidx = edge_index.astype(jnp.int32).reshape(-1)
    mul16 = jnp.broadcast_to(mul, (L,))
    return _graph_gather(tab8, eidx, mul16)


# speedup vs baseline: 1.0713x; 1.0713x over previous
import functools

import jax
import jax.numpy as jnp
from jax import lax
from jax.experimental import pallas as pl
from jax.experimental.pallas import tpu as pltpu
from jax.experimental.pallas import tpu_sc as plsc

NC = 2
NS = 16
L = 16
NW = NC * NS

N_POINTS = 100000
N_EDGES = 3200000
EPT = N_EDGES // NW
CH = 2000
NCHUNK = EPT // CH
RPS = N_POINTS // NS


@functools.partial(
    pl.kernel,
    out_type=(
        jax.ShapeDtypeStruct((N_EDGES, 3), jnp.float32),
        jax.ShapeDtypeStruct((N_EDGES, 3), jnp.float32),
    ),
    mesh=plsc.VectorSubcoreMesh(core_axis_name="c", subcore_axis_name="s"),
    scratch_types=[
        pltpu.VMEM((CH,), jnp.int32),
        pltpu.VMEM((CH,), jnp.int32),
        pltpu.VMEM((CH, 8), jnp.float32),
        pltpu.VMEM((CH, 8), jnp.float32),
        pltpu.VMEM((CH, 3), jnp.float32),
        pltpu.VMEM((L,), jnp.float32),
        pltpu.VMEM_SHARED((N_POINTS, 8), jnp.float32),
        pltpu.SemaphoreType.DMA,
        pltpu.SemaphoreType.DMA,
    ],
    compiler_params=pltpu.CompilerParams(
        use_tc_tiling_on_sc=False, needs_layout_passes=False
    ),
)
def _graph_gather(tab_hbm, eidx_hbm, mul_hbm, p1_hbm, p2_hbm,
                  idx_v0, idx_v1, rows8_0, rows8_1, rows3, mulv, tab_sh,
                  sem0, sem1):
    cid = lax.axis_index("c")
    sid = lax.axis_index("s")
    wid = sid * NC + cid

    pltpu.sync_copy(tab_hbm.at[pl.ds(sid * RPS, RPS)],
                    tab_sh.at[pl.ds(sid * RPS, RPS)])
    pltpu.sync_copy(mul_hbm, mulv)
    plsc.subcore_barrier()
    m = mulv[...]

    lanes = lax.iota(jnp.int32, 16)
    pats = []
    for j in range(3):
        p = j * 16 + lanes
        pats.append((p // 3, p - (p // 3) * 3))

    def compact_to(rows8):
        def compact(t, carry2):
            for j in range(3):
                r0, c0 = pats[j]
                r = t * 16 + r0
                v = plsc.load_gather(rows8, [r, c0])
                plsc.store_scatter(rows3, [r, c0], v * m)
            return carry2

        lax.fori_loop(0, CH // 16, compact, 0)

    def start_gather(e, g, idx_v, rows8, sem):
        base = wid * EPT + g * CH
        pltpu.sync_copy(eidx_hbm.at[pl.ds(e * N_EDGES + base, CH)], idx_v)
        pltpu.async_copy(tab_sh.at[idx_v], rows8, sem)

    for e, out_hbm in ((0, p1_hbm), (1, p2_hbm)):
        start_gather(e, 0, idx_v0, rows8_0, sem0)

        def pair_body(t, carry, e=e, out_hbm=out_hbm):
            g0 = 2 * t
            start_gather(e, g0 + 1, idx_v1, rows8_1, sem1)
            pltpu.make_async_copy(tab_sh.at[idx_v0], rows8_0, sem0).wait()
            compact_to(rows8_0)
            pltpu.sync_copy(rows3, out_hbm.at[pl.ds(wid * EPT + g0 * CH, CH)])

            @pl.when(g0 + 2 < NCHUNK)
            def _():
                start_gather(e, g0 + 2, idx_v0, rows8_0, sem0)

            pltpu.make_async_copy(tab_sh.at[idx_v1], rows8_1, sem1).wait()
            compact_to(rows8_1)
            pltpu.sync_copy(
                rows3, out_hbm.at[pl.ds(wid * EPT + (g0 + 1) * CH, CH)])
            return carry

        lax.fori_loop(0, NCHUNK // 2, pair_body, 0)


def kernel(coords, edge_index, mul):
    tab8 = jnp.pad(coords, ((0, 0), (0, 5)))
    eidx = edge_index.astype(jnp.int32).reshape(-1)
    mul16 = jnp.broadcast_to(mul, (L,))
    return _graph_gather(tab8, eidx, mul16)
